# baseline (speedup 1.0000x reference)
import numpy as np
import jax
import jax.numpy as jnp
from jax import lax
from jax.experimental import pallas as pl
from jax.experimental.pallas import tpu as pltpu
from jax.experimental.pallas import tpu_sc as plsc

D_MODEL = 1024
LENGTH = 2048
BATCH = 4
SEQ = 2048

NC, NS = 2, 16
NW = NC * NS
B_TOTAL = BATCH * SEQ
B_PER_W = B_TOTAL // NW
CHUNK = 32
N_CHUNKS = B_PER_W // CHUNK
LANES = 16
SCALE = 32.0


def _positional_encoding_np(length, depth):
    depth = depth / 2
    positions = np.arange(length)[:, np.newaxis]
    depths = np.arange(depth)[np.newaxis, :] / depth
    with np.errstate(divide="ignore", invalid="ignore"):
        angle_rates = 1 / (10000 * depths)
        angle_rads = positions * angle_rates
    return np.concatenate(
        [np.sin(angle_rads), np.cos(angle_rads)], axis=-1
    ).astype(np.float32)


_PE = _positional_encoding_np(LENGTH, D_MODEL)


def _sc_body(table_hbm, idx_hbm, pe_hbm, out_hbm, idx_v, rows_v, pe_v, sem):
    wid = lax.axis_index("s") * NC + lax.axis_index("c")
    base = wid * B_PER_W
    pbase = base % SEQ

    def chunk_step(c, carry):
        off = base + c * CHUNK
        poff = pbase + c * CHUNK
        pltpu.sync_copy(idx_hbm.at[pl.ds(off, CHUNK)], idx_v)
        gather = pltpu.async_copy(table_hbm.at[idx_v], rows_v, sem)
        pltpu.sync_copy(pe_hbm.at[pl.ds(poff, CHUNK)], pe_v)
        gather.wait()

        def row_step(r, carry2):
            for j in range(D_MODEL // LANES):
                s = pl.ds(j * LANES, LANES)
                rows_v[r, s] = rows_v[r, s] * SCALE + pe_v[r, s]
            return carry2

        lax.fori_loop(0, CHUNK, row_step, 0)
        pltpu.sync_copy(rows_v, out_hbm.at[pl.ds(off, CHUNK)])
        return carry

    lax.fori_loop(0, N_CHUNKS, chunk_step, 0)


@jax.jit
def _pos_embed(x_flat, table, pe):
    mesh = plsc.VectorSubcoreMesh(core_axis_name="c", subcore_axis_name="s")
    fn = pl.kernel(
        _sc_body,
        out_type=jax.ShapeDtypeStruct((B_TOTAL, D_MODEL), jnp.float32),
        mesh=mesh,
        scratch_types=[
            pltpu.VMEM((CHUNK,), jnp.int32),
            pltpu.VMEM((CHUNK, D_MODEL), jnp.float32),
            pltpu.VMEM((CHUNK, D_MODEL), jnp.float32),
            pltpu.SemaphoreType.DMA,
        ],
    )
    return fn(table, x_flat, pe)


def kernel(x, table):
    x_flat = x.reshape(B_TOTAL).astype(jnp.int32)
    pe = jnp.asarray(_PE)
    out = _pos_embed(x_flat, table, pe)
    return out.reshape(BATCH, SEQ, D_MODEL)


# speedup vs baseline: 1.2860x; 1.2860x over previous
import numpy as np
import jax
import jax.numpy as jnp
from jax import lax
from jax.experimental import pallas as pl
from jax.experimental.pallas import tpu as pltpu
from jax.experimental.pallas import tpu_sc as plsc

D_MODEL = 1024
LENGTH = 2048
BATCH = 4
SEQ = 2048

NC, NS = 2, 16
NW = NC * NS
POS_PER_W = SEQ // NW
CHUNK = 32
N_PC = POS_PER_W // CHUNK
N_IT = N_PC * BATCH
B_TOTAL = BATCH * SEQ
LANES = 16
SCALE = 32.0


def _positional_encoding_np(length, depth):
    depth = depth / 2
    positions = np.arange(length)[:, np.newaxis]
    depths = np.arange(depth)[np.newaxis, :] / depth
    with np.errstate(divide="ignore", invalid="ignore"):
        angle_rates = 1 / (10000 * depths)
        angle_rads = positions * angle_rates
    return np.concatenate(
        [np.sin(angle_rads), np.cos(angle_rads)], axis=-1
    ).astype(np.float32)


_PE = _positional_encoding_np(LENGTH, D_MODEL)


def _sc_body(table_hbm, idx_hbm, pe_hbm, out_hbm,
             idx_v, rows0, rows1, pe_v, sg0, sg1, ss0, ss1):
    wid = lax.axis_index("s") * NC + lax.axis_index("c")
    p0 = wid * POS_PER_W
    rows, sg, ss = [rows0, rows1], [sg0, sg1], [ss0, ss1]

    for b in range(BATCH):
        pltpu.sync_copy(idx_hbm.at[pl.ds(b * SEQ + p0, POS_PER_W)],
                        idx_v.at[pl.ds(b * POS_PER_W, POS_PER_W)])

    def offs(it):
        pc, b = it // BATCH, it % BATCH
        hbm_off = b * SEQ + p0 + pc * CHUNK
        idx_off = b * POS_PER_W + pc * CHUNK
        return hbm_off, idx_off

    gathers = [None, None]
    stores = [None, None]
    _, i0 = offs(0)
    gathers[0] = pltpu.async_copy(
        table_hbm.at[idx_v.at[pl.ds(i0, CHUNK)]], rows[0], sg[0])

    for it in range(N_IT):
        k = it % 2
        if it + 1 < N_IT:
            kn = (it + 1) % 2
            if stores[kn] is not None:
                stores[kn].wait()
            _, i1 = offs(it + 1)
            gathers[kn] = pltpu.async_copy(
                table_hbm.at[idx_v.at[pl.ds(i1, CHUNK)]], rows[kn], sg[kn])
        if it % BATCH == 0:
            pltpu.sync_copy(
                pe_hbm.at[pl.ds(p0 + (it // BATCH) * CHUNK, CHUNK)], pe_v)
        gathers[k].wait()

        def row_step(r, carry, _k=k):
            for j in range(D_MODEL // LANES):
                s = pl.ds(j * LANES, LANES)
                rows[_k][r, s] = rows[_k][r, s] * SCALE + pe_v[r, s]
            return carry

        lax.fori_loop(0, CHUNK, row_step, 0)
        o, _ = offs(it)
        stores[k] = pltpu.async_copy(
            rows[k], out_hbm.at[pl.ds(o, CHUNK)], ss[k])

    stores[0].wait()
    stores[1].wait()


@jax.jit
def _pos_embed(x_flat, table, pe):
    mesh = plsc.VectorSubcoreMesh(core_axis_name="c", subcore_axis_name="s")
    fn = pl.kernel(
        _sc_body,
        out_type=jax.ShapeDtypeStruct((B_TOTAL, D_MODEL), jnp.float32),
        mesh=mesh,
        scratch_types=[
            pltpu.VMEM((BATCH * POS_PER_W,), jnp.int32),
            pltpu.VMEM((CHUNK, D_MODEL), jnp.float32),
            pltpu.VMEM((CHUNK, D_MODEL), jnp.float32),
            pltpu.VMEM((CHUNK, D_MODEL), jnp.float32),
            pltpu.SemaphoreType.DMA,
            pltpu.SemaphoreType.DMA,
            pltpu.SemaphoreType.DMA,
            pltpu.SemaphoreType.DMA,
        ],
    )
    return fn(table, x_flat, pe)


def kernel(x, table):
    x_flat = x.reshape(B_TOTAL).astype(jnp.int32)
    pe = jnp.asarray(_PE)
    out = _pos_embed(x_flat, table, pe)
    return out.reshape(BATCH, SEQ, D_MODEL)


# speedup vs baseline: 1.3802x; 1.0733x over previous
import numpy as np
import jax
import jax.numpy as jnp
from jax import lax
from jax.experimental import pallas as pl
from jax.experimental.pallas import tpu as pltpu
from jax.experimental.pallas import tpu_sc as plsc

D_MODEL = 1024
LENGTH = 2048
BATCH = 4
SEQ = 2048

NC, NS = 2, 16
NW = NC * NS
POS_PER_W = SEQ // NW
CHUNK = 16
N_PC = POS_PER_W // CHUNK
N_IT = N_PC * BATCH
RING = 6
AHEAD = 3
LANES = 16
SCALE = 32.0


def _positional_encoding_np(length, depth):
    depth = depth / 2
    positions = np.arange(length)[:, np.newaxis]
    depths = np.arange(depth)[np.newaxis, :] / depth
    with np.errstate(divide="ignore", invalid="ignore"):
        angle_rates = 1 / (10000 * depths)
        angle_rads = positions * angle_rates
    return np.concatenate(
        [np.sin(angle_rads), np.cos(angle_rads)], axis=-1
    ).astype(np.float32)


_PE = _positional_encoding_np(LENGTH, D_MODEL)


def _sc_body(table_hbm, idx_hbm, pe_hbm, out_hbm,
             idx_v, r0, r1, r2, r3, r4, r5, pe_v, sg, ss):
    wid = lax.axis_index("s") * NC + lax.axis_index("c")
    p0 = wid * POS_PER_W
    rows = [r0, r1, r2, r3, r4, r5]

    for b in range(BATCH):
        pltpu.sync_copy(idx_hbm.at[b, pl.ds(p0, POS_PER_W)],
                        idx_v.at[pl.ds(b * POS_PER_W, POS_PER_W)])

    def offs(it):
        pc, b = it // BATCH, it % BATCH
        return b, p0 + pc * CHUNK, b * POS_PER_W + pc * CHUNK

    def start_gather(it):
        _, _, idx_off = offs(it)
        k = it % RING
        return pltpu.async_copy(
            table_hbm.at[idx_v.at[pl.ds(idx_off, CHUNK)]], rows[k], sg.at[k])

    gathers = [None] * N_IT
    stores = [None] * N_IT
    for it in range(AHEAD):
        gathers[it] = start_gather(it)

    for it in range(N_IT):
        k = it % RING
        nxt = it + AHEAD
        if nxt < N_IT:
            if nxt - RING >= 0:
                stores[nxt - RING].wait()
            gathers[nxt] = start_gather(nxt)
        if it % BATCH == 0:
            pltpu.sync_copy(
                pe_hbm.at[pl.ds(p0 + (it // BATCH) * CHUNK, CHUNK)], pe_v)
        gathers[it].wait()

        def row_step(r, carry, _k=k):
            for j in range(D_MODEL // LANES):
                s = pl.ds(j * LANES, LANES)
                rows[_k][r, s] = rows[_k][r, s] * SCALE + pe_v[r, s]
            return carry

        lax.fori_loop(0, CHUNK, row_step, 0)
        b, pos, _ = offs(it)
        stores[it] = pltpu.async_copy(
            rows[k], out_hbm.at[b, pl.ds(pos, CHUNK)], ss.at[k])

    for it in range(N_IT - RING, N_IT):
        stores[it].wait()


@jax.jit
def _pos_embed(x2d, table, pe):
    mesh = plsc.VectorSubcoreMesh(core_axis_name="c", subcore_axis_name="s")
    fn = pl.kernel(
        _sc_body,
        out_type=jax.ShapeDtypeStruct((BATCH, SEQ, D_MODEL), jnp.float32),
        mesh=mesh,
        scratch_types=[
            pltpu.VMEM((BATCH * POS_PER_W,), jnp.int32),
        ] + [pltpu.VMEM((CHUNK, D_MODEL), jnp.float32)] * (RING + 1) + [
            pltpu.SemaphoreType.DMA((RING,)),
            pltpu.SemaphoreType.DMA((RING,)),
        ],
    )
    return fn(table, x2d, pe)


def kernel(x, table):
    x2d = x.astype(jnp.int32)
    pe = jnp.asarray(_PE)
    return _pos_embed(x2d, table, pe)
